# Initial kernel scaffold; baseline (speedup 1.0000x reference)
import jax
import jax.numpy as jnp
from jax import lax
from jax.experimental import pallas as pl
from jax.experimental.pallas import tpu as pltpu
from jax.experimental.pallas import tpu_sc as plsc

N_NODES = 10000
N_EDGES = 320000
D_IN = 128
D_HID = 128
D_OUT = 112

NC = 2
NS = 16
NW = NC * NS
EDGES_PER_W = N_EDGES // NW
CHUNK = 128
NFULL = EDGES_PER_W // CHUNK
TAIL = EDGES_PER_W - NFULL * CHUNK
ROWS_PER_S = N_NODES // NS
ZCH = 125
CNT_W = 16


def _fill(ref, nrows, ncols, val):
    v = jnp.full((16,), val, jnp.float32)

    def body(r, _):
        for j in range(ncols // 16):
            ref[r, pl.ds(j * 16, 16)] = v
        return 0

    lax.fori_loop(0, nrows, body, 0)


def _make_agg(with_counts):
    out_type = [jax.ShapeDtypeStruct((NC, N_NODES, D_HID), jnp.float32)]
    if with_counts:
        out_type.append(jax.ShapeDtypeStruct((NC, N_NODES, CNT_W), jnp.float32))

    scratch = [
        pltpu.VMEM((CHUNK,), jnp.int32),
        pltpu.VMEM((CHUNK,), jnp.int32),
        pltpu.VMEM((TAIL,), jnp.int32),
        pltpu.VMEM((TAIL,), jnp.int32),
        pltpu.VMEM((CHUNK, D_HID), jnp.float32),
        pltpu.VMEM((TAIL, D_HID), jnp.float32),
        pltpu.VMEM_SHARED((N_NODES, D_HID), jnp.float32),
        pltpu.SemaphoreType.DMA,
    ]
    if with_counts:
        scratch += [
            pltpu.VMEM((CHUNK, CNT_W), jnp.float32),
            pltpu.VMEM((TAIL, CNT_W), jnp.float32),
            pltpu.VMEM((ZCH, CNT_W), jnp.float32),
            pltpu.VMEM_SHARED((N_NODES, CNT_W), jnp.float32),
        ]

    def body(table, src, dst, *rest):
        if with_counts:
            (sum_out, cnt_out, src_v, dst_v, src_t, dst_t, rows_v, rows_t,
             acc, gsem, ones_v, ones_t, cz_v, acc_cnt) = rest
        else:
            (sum_out, src_v, dst_v, src_t, dst_t, rows_v, rows_t,
             acc, gsem) = rest

        cid = lax.axis_index("c")
        sid = lax.axis_index("s")
        wid = cid * NS + sid

        _fill(rows_v, CHUNK, D_HID, 0.0)
        rbase = sid * ROWS_PER_S
        for i in range(ROWS_PER_S // ZCH):
            pltpu.sync_copy(rows_v.at[pl.ds(0, ZCH)],
                            acc.at[pl.ds(rbase + i * ZCH, ZCH)])
        if with_counts:
            _fill(cz_v, ZCH, CNT_W, 0.0)
            _fill(ones_v, CHUNK, CNT_W, 1.0)
            _fill(ones_t, TAIL, CNT_W, 1.0)
            for i in range(ROWS_PER_S // ZCH):
                pltpu.sync_copy(cz_v, acc_cnt.at[pl.ds(rbase + i * ZCH, ZCH)])
        plsc.subcore_barrier()

        ebase = wid * EDGES_PER_W

        def chunk_body(c, _):
            off = ebase + c * CHUNK
            pltpu.sync_copy(src.at[pl.ds(off, CHUNK)], src_v)
            pltpu.sync_copy(dst.at[pl.ds(off, CHUNK)], dst_v)
            pltpu.async_copy(table.at[src_v], rows_v, gsem).wait()
            pltpu.sync_copy(rows_v, acc.at[dst_v], add=True)
            if with_counts:
                pltpu.sync_copy(ones_v, acc_cnt.at[dst_v], add=True)
            return 0

        lax.fori_loop(0, NFULL, chunk_body, 0)

        toff = ebase + NFULL * CHUNK
        pltpu.sync_copy(src.at[pl.ds(toff, TAIL)], src_t)
        pltpu.sync_copy(dst.at[pl.ds(toff, TAIL)], dst_t)
        pltpu.async_copy(table.at[src_t], rows_t, gsem).wait()
        pltpu.sync_copy(rows_t, acc.at[dst_t], add=True)
        if with_counts:
            pltpu.sync_copy(ones_t, acc_cnt.at[dst_t], add=True)

        plsc.subcore_barrier()

        for i in range(ROWS_PER_S // ZCH):
            roff = rbase + i * ZCH
            pltpu.sync_copy(acc.at[pl.ds(roff, ZCH)], rows_v.at[pl.ds(0, ZCH)])
            pltpu.sync_copy(rows_v.at[pl.ds(0, ZCH)],
                            sum_out.at[cid, pl.ds(roff, ZCH)])
            if with_counts:
                pltpu.sync_copy(acc_cnt.at[pl.ds(roff, ZCH)], cz_v)
                pltpu.sync_copy(cz_v, cnt_out.at[cid, pl.ds(roff, ZCH)])

    mesh = plsc.VectorSubcoreMesh(core_axis_name="c", subcore_axis_name="s")
    return pl.kernel(body, out_type=out_type, mesh=mesh,
                     scratch_types=scratch)


_agg_with_counts = _make_agg(True)
_agg_no_counts = _make_agg(False)


BN = 256


def _layer1_tc(sum_ref, cnt_ref, x_ref, wl_ref, bl_ref, wr_ref, out_ref):
    s = sum_ref[0] + sum_ref[1]
    c = cnt_ref[0, :, 0:1] + cnt_ref[1, :, 0:1]
    mean = s / jnp.maximum(c, 1.0)
    h = (jnp.dot(mean, wl_ref[...], preferred_element_type=jnp.float32)
         + jnp.dot(x_ref[...], wr_ref[...], preferred_element_type=jnp.float32)
         + bl_ref[...])
    out_ref[...] = jnp.maximum(h, 0.0)


def _layer2_tc(sum_ref, cnt_ref, h_ref, wl_ref, bl_ref, wr_ref, wo_ref,
               bo_ref, out_ref):
    s = sum_ref[0] + sum_ref[1]
    c = cnt_ref[0, :, 0:1] + cnt_ref[1, :, 0:1]
    mean = s / jnp.maximum(c, 1.0)
    h = (jnp.dot(mean, wl_ref[...], preferred_element_type=jnp.float32)
         + jnp.dot(h_ref[...], wr_ref[...], preferred_element_type=jnp.float32)
         + bl_ref[...])
    h = jnp.maximum(h, 0.0)
    out_ref[...] = (jnp.dot(h, wo_ref[...], preferred_element_type=jnp.float32)
                    + bo_ref[...])


def _row_block(d):
    return pl.BlockSpec((BN, d), lambda i: (i, 0))


def _part_block(d):
    return pl.BlockSpec((NC, BN, d), lambda i: (0, i, 0))


def _full_block(a, b):
    return pl.BlockSpec((a, b), lambda i: (0, 0))


_GRID = (pl.cdiv(N_NODES, BN),)

_layer1_call = pl.pallas_call(
    _layer1_tc,
    grid=_GRID,
    in_specs=[_part_block(D_HID), _part_block(CNT_W), _row_block(D_IN),
              _full_block(D_IN, D_HID), _full_block(1, D_HID),
              _full_block(D_IN, D_HID)],
    out_specs=_row_block(D_HID),
    out_shape=jax.ShapeDtypeStruct((N_NODES, D_HID), jnp.float32),
)

_layer2_call = pl.pallas_call(
    _layer2_tc,
    grid=_GRID,
    in_specs=[_part_block(D_HID), _part_block(CNT_W), _row_block(D_HID),
              _full_block(D_HID, D_HID), _full_block(1, D_HID),
              _full_block(D_HID, D_HID), _full_block(D_HID, D_OUT),
              _full_block(1, D_OUT)],
    out_specs=_row_block(D_OUT),
    out_shape=jax.ShapeDtypeStruct((N_NODES, D_OUT), jnp.float32),
)


def kernel(x, edge_index, W_l1, b_l1, W_r1, W_l2, b_l2, W_r2, W_out, b_out):
    ei = edge_index.astype(jnp.int32)
    src = ei[0]
    dst = ei[1]

    sum1, cnt = _agg_with_counts(x, src, dst)
    h1 = _layer1_call(sum1, cnt, x, W_l1, b_l1.reshape(1, D_HID), W_r1)

    (sum2,) = _agg_no_counts(h1, src, dst)
    out = _layer2_call(sum2, cnt, h1, W_l2, b_l2.reshape(1, D_HID), W_r2,
                       W_out, b_out.reshape(1, D_OUT))
    return out


# speedup vs baseline: 4.0117x; 4.0117x over previous
import jax
import jax.numpy as jnp
from jax import lax
from jax.experimental import pallas as pl
from jax.experimental.pallas import tpu as pltpu
from jax.experimental.pallas import tpu_sc as plsc

N_NODES = 10000
N_EDGES = 320000
D_IN = 128
D_HID = 128
D_OUT = 112
D = 128

NC = 2
NS = 16
NW = NC * NS
NA = 10240
TRASH = 10200
RCH = 64
RCHUNKS = 10
CHUNK = 96
NCHUNKS = 105
EPW = CHUNK * NCHUNKS
E_PAD = EPW * NW


def _fill(ref, nrows, ncols, val):
    v = jnp.full((16,), val, jnp.float32)

    def body(r, _):
        for j in range(ncols // 16):
            ref[r, pl.ds(j * 16, 16)] = v
        return 0

    lax.fori_loop(0, nrows, body, 0)


def _make_agg(with_counts):
    out_type = [jax.ShapeDtypeStruct((NC, NA, D), jnp.float32)]
    if with_counts:
        out_type.append(jax.ShapeDtypeStruct((NC, NA, D), jnp.float32))

    scratch = [
        pltpu.VMEM((CHUNK,), jnp.int32),
        pltpu.VMEM((CHUNK,), jnp.int32),
        pltpu.VMEM((CHUNK, D), jnp.float32),
        pltpu.VMEM_SHARED((NA, D), jnp.float32),
        pltpu.SemaphoreType.DMA,
    ]

    def body(table, src, dst, *rest):
        if with_counts:
            (sum_out, cnt_out, src_v, dst_v, rows_v, acc, gsem) = rest
        else:
            (sum_out, src_v, dst_v, rows_v, acc, gsem) = rest

        sid = lax.axis_index("s")
        cid = lax.axis_index("c")
        wid = cid * NS + sid
        ebase = wid * EPW

        def zero_acc():
            for k in range(RCHUNKS):
                roff = (sid * RCHUNKS + k) * RCH
                pltpu.sync_copy(rows_v.at[pl.ds(0, RCH)],
                                acc.at[pl.ds(roff, RCH)])

        def write_acc(out):
            for k in range(RCHUNKS):
                roff = (sid * RCHUNKS + k) * RCH
                pltpu.sync_copy(acc.at[pl.ds(roff, RCH)],
                                rows_v.at[pl.ds(0, RCH)])
                pltpu.sync_copy(rows_v.at[pl.ds(0, RCH)],
                                out.at[cid, pl.ds(roff, RCH)])

        _fill(rows_v, CHUNK, D, 0.0)
        zero_acc()
        plsc.subcore_barrier()

        if with_counts:
            _fill(rows_v, CHUNK, D, 1.0)

            def cnt_body(c, _):
                off = ebase + c * CHUNK
                pltpu.sync_copy(dst.at[pl.ds(off, CHUNK)], dst_v)
                pltpu.sync_copy(rows_v, acc.at[dst_v], add=True)
                return 0

            lax.fori_loop(0, NCHUNKS, cnt_body, 0)
            plsc.subcore_barrier()
            write_acc(cnt_out)
            _fill(rows_v, CHUNK, D, 0.0)
            zero_acc()
            plsc.subcore_barrier()

        def chunk_body(c, _):
            off = ebase + c * CHUNK
            pltpu.sync_copy(src.at[pl.ds(off, CHUNK)], src_v)
            pltpu.sync_copy(dst.at[pl.ds(off, CHUNK)], dst_v)
            pltpu.async_copy(table.at[src_v], rows_v, gsem).wait()
            pltpu.sync_copy(rows_v, acc.at[dst_v], add=True)
            return 0

        lax.fori_loop(0, NCHUNKS, chunk_body, 0)
        plsc.subcore_barrier()
        write_acc(sum_out)

    mesh = plsc.VectorSubcoreMesh(core_axis_name="c", subcore_axis_name="s")
    return pl.kernel(body, out_type=out_type, mesh=mesh,
                     scratch_types=scratch)


_agg_with_counts = _make_agg(True)
_agg_no_counts = _make_agg(False)


BN = 256


def _layer1_tc(sum_ref, inv_ref, x_ref, wl_ref, bl_ref, wr_ref, out_ref):
    mean = (sum_ref[0] + sum_ref[1]) * inv_ref[...]
    h = (jnp.dot(mean, wl_ref[...], preferred_element_type=jnp.float32)
         + jnp.dot(x_ref[...], wr_ref[...], preferred_element_type=jnp.float32)
         + bl_ref[...])
    out_ref[...] = jnp.maximum(h, 0.0)


def _layer2_tc(sum_ref, inv_ref, h_ref, wl_ref, bl_ref, wr_ref, wo_ref,
               bo_ref, out_ref):
    mean = (sum_ref[0] + sum_ref[1]) * inv_ref[...]
    h = (jnp.dot(mean, wl_ref[...], preferred_element_type=jnp.float32)
         + jnp.dot(h_ref[...], wr_ref[...], preferred_element_type=jnp.float32)
         + bl_ref[...])
    h = jnp.maximum(h, 0.0)
    out_ref[...] = (jnp.dot(h, wo_ref[...], preferred_element_type=jnp.float32)
                    + bo_ref[...])


def _row_block(d):
    return pl.BlockSpec((BN, d), lambda i: (i, 0))


def _part_block(d):
    return pl.BlockSpec((NC, BN, d), lambda i: (0, i, 0))


def _full_block(a, b):
    return pl.BlockSpec((a, b), lambda i: (0, 0))


_GRID = (pl.cdiv(N_NODES, BN),)

_layer1_call = pl.pallas_call(
    _layer1_tc,
    grid=_GRID,
    in_specs=[_part_block(D), _row_block(1), _row_block(D_IN),
              _full_block(D_IN, D_HID), _full_block(1, D_HID),
              _full_block(D_IN, D_HID)],
    out_specs=_row_block(D_HID),
    out_shape=jax.ShapeDtypeStruct((N_NODES, D_HID), jnp.float32),
)

_layer2_call = pl.pallas_call(
    _layer2_tc,
    grid=_GRID,
    in_specs=[_part_block(D), _row_block(1), _row_block(D_HID),
              _full_block(D_HID, D_HID), _full_block(1, D_HID),
              _full_block(D_HID, D_HID), _full_block(D_HID, D_OUT),
              _full_block(1, D_OUT)],
    out_specs=_row_block(D_OUT),
    out_shape=jax.ShapeDtypeStruct((N_NODES, D_OUT), jnp.float32),
)


def kernel(x, edge_index, W_l1, b_l1, W_r1, W_l2, b_l2, W_r2, W_out, b_out):
    ei = edge_index.astype(jnp.int32)
    npad = E_PAD - N_EDGES
    src_p = jnp.concatenate([ei[0], jnp.zeros((npad,), jnp.int32)])
    dst_p = jnp.concatenate([ei[1], jnp.full((npad,), TRASH, jnp.int32)])

    sum1, cnt = _agg_with_counts(x, src_p, dst_p)
    inv = (1.0 / jnp.clip(cnt[0, :, 0] + cnt[1, :, 0], 1.0, None))
    inv = inv.reshape(NA, 1)
    h1 = _layer1_call(sum1, inv, x, W_l1, b_l1.reshape(1, D_HID), W_r1)

    (sum2,) = _agg_no_counts(h1, src_p, dst_p)
    out = _layer2_call(sum2, inv, h1, W_l2, b_l2.reshape(1, D_HID), W_r2,
                       W_out, b_out.reshape(1, D_OUT))
    return out
